# baseline (speedup 1.0000x reference)
import functools
import jax
import jax.numpy as jnp
from jax import lax
from jax.experimental import pallas as pl
from jax.experimental.pallas import tpu as pltpu
from jax.experimental.pallas import tpu_sc as plsc

_N = 10000
_E = 320000
_D = 128

_NC = 2
_NS = 16
_NW = _NC * _NS
_CHUNK = 128
_EPW = 10240
_EPAD = _NW * _EPW
_NCHUNK = _EPW // _CHUNK
_K = 40
_NSUP = _NCHUNK // _K
_NPAD = 10240
_RPW = _NPAD // _NS


def _sc_aggr_body(h_hbm, src_hbm, dst_hbm, zro_hbm, out_hbm,
                  idx_s, idx_d, rows0, rows1, acc, sem0, sem1, ssem0, ssem1):
    c = lax.axis_index("c")
    s = lax.axis_index("s")
    wid = c * _NS + s
    pltpu.sync_copy(zro_hbm, rows0)
    for r in range(_RPW // _CHUNK):
        pltpu.sync_copy(rows0, acc.at[pl.ds(s * _RPW + r * _CHUNK, _CHUNK)])
    plsc.subcore_barrier()

    def gather(j, buf, sem):
        return pltpu.async_copy(h_hbm.at[idx_d.at[j]], buf, sem)

    def gather_wait(j, buf, sem):
        pltpu.make_async_copy(h_hbm.at[idx_d.at[j]], buf, sem).wait()

    def scat(j, buf):
        pltpu.sync_copy(buf, acc.at[idx_s.at[j]], add=True)

    def scat_start(j, buf, sem):
        pltpu.async_copy(buf, acc.at[idx_s.at[j]], sem, add=True)

    def scat_wait(j, buf, sem):
        pltpu.make_async_copy(buf, acc.at[idx_s.at[j]], sem).wait()

    @pl.loop(0, _NSUP)
    def _(p):
        pltpu.sync_copy(src_hbm.at[wid, p], idx_s)
        pltpu.sync_copy(dst_hbm.at[wid, p], idx_d)
        gather(0, rows0, sem0)
        gather(1, rows1, sem1)

        @pl.loop(0, _K - 2, step=2)
        def _(j):
            gather_wait(j, rows0, sem0)
            scat_start(j, rows0, ssem0)
            gather_wait(j + 1, rows1, sem1)
            scat_start(j + 1, rows1, ssem1)
            scat_wait(j, rows0, ssem0)
            gather(j + 2, rows0, sem0)
            scat_wait(j + 1, rows1, ssem1)
            gather(j + 3, rows1, sem1)

        gather_wait(_K - 2, rows0, sem0)
        scat(_K - 2, rows0)
        gather_wait(_K - 1, rows1, sem1)
        scat(_K - 1, rows1)

    plsc.subcore_barrier()
    pltpu.sync_copy(acc.at[pl.ds(s * _RPW, _RPW)], out_hbm.at[wid])


_sc_aggr = pl.kernel(
    _sc_aggr_body,
    out_type=jax.ShapeDtypeStruct((_NW, _RPW, _D), jnp.float32),
    mesh=plsc.VectorSubcoreMesh(core_axis_name="c", subcore_axis_name="s"),
    scratch_types=[
        pltpu.VMEM((_K, _CHUNK), jnp.int32),
        pltpu.VMEM((_K, _CHUNK), jnp.int32),
        pltpu.VMEM((_CHUNK, _D), jnp.float32),
        pltpu.VMEM((_CHUNK, _D), jnp.float32),
        pltpu.VMEM_SHARED((_NPAD, _D), jnp.float32),
        pltpu.SemaphoreType.DMA,
        pltpu.SemaphoreType.DMA,
        pltpu.SemaphoreType.DMA,
        pltpu.SemaphoreType.DMA,
    ],
    name="sc_edge_aggr",
)

_BLK = _RPW
_GRID = _NPAD // _BLK


def _tc_layer_body(p0, p1, h, wl, w0, w1, b, o):
    aggr = p0[0] + p1[0]
    acc = jnp.dot(aggr, wl[...], preferred_element_type=jnp.float32)
    acc += jnp.dot(h[...], w0[...] + w1[...], preferred_element_type=jnp.float32)
    o[...] = jnp.maximum(acc + b[...], 0.0)


def _tc_final_body(p0, p1, h, wl, w0, w1, b, wout, bout, o):
    aggr = p0[0] + p1[0]
    acc = jnp.dot(aggr, wl[...], preferred_element_type=jnp.float32)
    acc += jnp.dot(h[...], w0[...] + w1[...], preferred_element_type=jnp.float32)
    hid = jnp.maximum(acc + b[...], 0.0)
    o[...] = jnp.dot(hid, wout[...], preferred_element_type=jnp.float32) + bout[...]


_row_spec = pl.BlockSpec((_BLK, _D), lambda i: (i, 0))
_p0_spec = pl.BlockSpec((1, _RPW, _D), lambda i: (i, 0, 0))
_p1_spec = pl.BlockSpec((1, _RPW, _D), lambda i: (i + _NS, 0, 0))
_w_spec = pl.BlockSpec((_D, _D), lambda i: (0, 0))
_b_spec = pl.BlockSpec((1, _D), lambda i: (0, 0))

_tc_layer = pl.pallas_call(
    _tc_layer_body,
    grid=(_GRID,),
    in_specs=[_p0_spec, _p1_spec, _row_spec, _w_spec, _w_spec, _w_spec, _b_spec],
    out_specs=_row_spec,
    out_shape=jax.ShapeDtypeStruct((_NPAD, _D), jnp.float32),
)

_tc_final = pl.pallas_call(
    _tc_final_body,
    grid=(_GRID,),
    in_specs=[_p0_spec, _p1_spec, _row_spec, _w_spec, _w_spec, _w_spec,
              _b_spec, _w_spec, _b_spec],
    out_specs=_row_spec,
    out_shape=jax.ShapeDtypeStruct((_NPAD, _D), jnp.float32),
)


@jax.jit
def kernel(x, edge_index_0, edge_index_1,
           W0_0, b0_0, Wl_0, bl_0, W1_0, b1_0,
           W0_1, b0_1, Wl_1, bl_1, W1_1, b1_1,
           Wout, bout):
    zeros = jnp.zeros((_CHUNK, _D), jnp.float32)
    xp = jnp.pad(x, ((0, _NPAD - _N), (0, 0)))

    npad = _EPAD - _E
    spad = _N + (jnp.arange(npad, dtype=jnp.int32) % (_NPAD - _N))
    dpad = jnp.arange(npad, dtype=jnp.int32) % _N

    def edge_views(ei):
        src = jnp.concatenate([ei[0], spad]).reshape(_NW, _NSUP, _K, _CHUNK)
        dst = jnp.concatenate([ei[1], dpad]).reshape(_NW, _NSUP, _K, _CHUNK)
        return src, dst

    src1, dst1 = edge_views(edge_index_1)
    src0, dst0 = edge_views(edge_index_0)

    P = _sc_aggr(xp, src1, dst1, zeros)
    h1 = _tc_layer(P, P, xp, Wl_1, W0_1, W1_1,
                   (bl_1 + b0_1 + b1_1).reshape(1, _D))

    Q = _sc_aggr(h1, src0, dst0, zeros)
    out = _tc_final(Q, Q, h1, Wl_0, W0_0, W1_0,
                    (bl_0 + b0_0 + b1_0).reshape(1, _D),
                    Wout, bout.reshape(1, _D))
    return out[:_N]


# speedup vs baseline: 1.2503x; 1.2503x over previous
import functools
import jax
import jax.numpy as jnp
from jax import lax
from jax.experimental import pallas as pl
from jax.experimental.pallas import tpu as pltpu
from jax.experimental.pallas import tpu_sc as plsc

_N = 10000
_E = 320000
_D = 128

_NC = 2
_NS = 16
_NW = _NC * _NS
_CHUNK = 128
_EPW = 10240
_EPAD = _NW * _EPW
_NCHUNK = _EPW // _CHUNK
_K = 40
_NSUP = _NCHUNK // _K
_NPAD = 10240
_RPW = _NPAD // _NS


def _sc_aggr_body(h_hbm, src_hbm, dst_hbm, zro_hbm, out_hbm,
                  idx_s, idx_d, rows0, rows1, acc, sem0, sem1):
    c = lax.axis_index("c")
    s = lax.axis_index("s")
    wid = c * _NS + s
    pltpu.sync_copy(src_hbm.at[wid, 0], idx_s)
    pltpu.sync_copy(dst_hbm.at[wid, 0], idx_d)
    pltpu.async_copy(h_hbm.at[idx_d.at[0]], rows0, sem0)
    pltpu.sync_copy(zro_hbm, rows1)
    for r in range(_RPW // _CHUNK):
        pltpu.async_copy(rows1, acc.at[pl.ds(s * _RPW + r * _CHUNK, _CHUNK)],
                         sem1)
    for r in range(_RPW // _CHUNK):
        pltpu.make_async_copy(rows1, acc.at[pl.ds(s * _RPW + r * _CHUNK,
                                                  _CHUNK)], sem1).wait()
    plsc.subcore_barrier()

    def gather(j, buf, sem):
        return pltpu.async_copy(h_hbm.at[idx_d.at[j]], buf, sem)

    def gather_wait(j, buf, sem):
        pltpu.make_async_copy(h_hbm.at[idx_d.at[j]], buf, sem).wait()

    def scat(j, buf):
        pltpu.sync_copy(buf, acc.at[idx_s.at[j]], add=True)

    @pl.loop(0, _NSUP)
    def _(p):
        @pl.when(p > 0)
        def _():
            pltpu.sync_copy(src_hbm.at[wid, p], idx_s)
            pltpu.sync_copy(dst_hbm.at[wid, p], idx_d)
            gather(0, rows0, sem0)

        gather(1, rows1, sem1)

        @pl.loop(0, _K - 2, step=2)
        def _(j):
            gather_wait(j, rows0, sem0)
            scat(j, rows0)
            gather(j + 2, rows0, sem0)
            gather_wait(j + 1, rows1, sem1)
            scat(j + 1, rows1)
            gather(j + 3, rows1, sem1)

        gather_wait(_K - 2, rows0, sem0)
        scat(_K - 2, rows0)
        gather_wait(_K - 1, rows1, sem1)
        scat(_K - 1, rows1)

    plsc.subcore_barrier()
    pltpu.sync_copy(acc.at[pl.ds(s * _RPW, _RPW)], out_hbm.at[wid])


_sc_aggr = pl.kernel(
    _sc_aggr_body,
    out_type=jax.ShapeDtypeStruct((_NW, _RPW, _D), jnp.float32),
    mesh=plsc.VectorSubcoreMesh(core_axis_name="c", subcore_axis_name="s"),
    scratch_types=[
        pltpu.VMEM((_K, _CHUNK), jnp.int32),
        pltpu.VMEM((_K, _CHUNK), jnp.int32),
        pltpu.VMEM((_CHUNK, _D), jnp.float32),
        pltpu.VMEM((_CHUNK, _D), jnp.float32),
        pltpu.VMEM_SHARED((_NPAD, _D), jnp.float32),
        pltpu.SemaphoreType.DMA,
        pltpu.SemaphoreType.DMA,
    ],
    name="sc_edge_aggr",
)

_BLK = _RPW
_GRID = _NPAD // _BLK


def _tc_layer_body(p0, p1, h, wl, w0, w1, b, o):
    aggr = p0[0] + p1[0]
    acc = jnp.dot(aggr, wl[...], preferred_element_type=jnp.float32)
    acc += jnp.dot(h[...], w0[...] + w1[...], preferred_element_type=jnp.float32)
    o[...] = jnp.maximum(acc + b[...], 0.0)


def _tc_final_body(p0, p1, h, wl, w0, w1, b, wout, bout, o):
    aggr = p0[0] + p1[0]
    acc = jnp.dot(aggr, wl[...], preferred_element_type=jnp.float32)
    acc += jnp.dot(h[...], w0[...] + w1[...], preferred_element_type=jnp.float32)
    hid = jnp.maximum(acc + b[...], 0.0)
    o[...] = jnp.dot(hid, wout[...], preferred_element_type=jnp.float32) + bout[...]


_row_spec = pl.BlockSpec((_BLK, _D), lambda i: (i, 0))
_p0_spec = pl.BlockSpec((1, _RPW, _D), lambda i: (i, 0, 0))
_p1_spec = pl.BlockSpec((1, _RPW, _D), lambda i: (i + _NS, 0, 0))
_w_spec = pl.BlockSpec((_D, _D), lambda i: (0, 0))
_b_spec = pl.BlockSpec((1, _D), lambda i: (0, 0))

_tc_layer = pl.pallas_call(
    _tc_layer_body,
    grid=(_GRID,),
    in_specs=[_p0_spec, _p1_spec, _row_spec, _w_spec, _w_spec, _w_spec, _b_spec],
    out_specs=_row_spec,
    out_shape=jax.ShapeDtypeStruct((_NPAD, _D), jnp.float32),
)

_tc_final = pl.pallas_call(
    _tc_final_body,
    grid=(_GRID,),
    in_specs=[_p0_spec, _p1_spec, _row_spec, _w_spec, _w_spec, _w_spec,
              _b_spec, _w_spec, _b_spec],
    out_specs=_row_spec,
    out_shape=jax.ShapeDtypeStruct((_NPAD, _D), jnp.float32),
)


@jax.jit
def kernel(x, edge_index_0, edge_index_1,
           W0_0, b0_0, Wl_0, bl_0, W1_0, b1_0,
           W0_1, b0_1, Wl_1, bl_1, W1_1, b1_1,
           Wout, bout):
    zeros = jnp.zeros((_CHUNK, _D), jnp.float32)
    xp = jnp.pad(x, ((0, _NPAD - _N), (0, 0)))

    npad = _EPAD - _E
    spad = _N + (jnp.arange(npad, dtype=jnp.int32) % (_NPAD - _N))
    dpad = jnp.arange(npad, dtype=jnp.int32) % _N

    def edge_views(ei):
        src = jnp.concatenate([ei[0], spad]).reshape(_NW, _NSUP, _K, _CHUNK)
        dst = jnp.concatenate([ei[1], dpad]).reshape(_NW, _NSUP, _K, _CHUNK)
        return src, dst

    src1, dst1 = edge_views(edge_index_1)
    src0, dst0 = edge_views(edge_index_0)

    P = _sc_aggr(xp, src1, dst1, zeros)
    h1 = _tc_layer(P, P, xp, Wl_1, W0_1, W1_1,
                   (bl_1 + b0_1 + b1_1).reshape(1, _D))

    Q = _sc_aggr(h1, src0, dst0, zeros)
    out = _tc_final(Q, Q, h1, Wl_0, W0_0, W1_0,
                    (bl_0 + b0_0 + b1_0).reshape(1, _D),
                    Wout, bout.reshape(1, _D))
    return out[:_N]


# speedup vs baseline: 1.2915x; 1.0330x over previous
import functools
import jax
import jax.numpy as jnp
from jax import lax
from jax.experimental import pallas as pl
from jax.experimental.pallas import tpu as pltpu
from jax.experimental.pallas import tpu_sc as plsc

_N = 10000
_E = 320000
_D = 128

_NC = 2
_NS = 16
_NW = _NC * _NS
_CHUNK = 128
_EPW = 10240
_EPAD = _NW * _EPW
_NCHUNK = _EPW // _CHUNK
_K = 40
_NSUP = _NCHUNK // _K
_NPAD = 10240
_RPW = _NPAD // _NS


def _sc_aggr_body(h_hbm, src_hbm, dst_hbm, zro_hbm, out_hbm,
                  idx_s, idx_d, rows0, rows1, acc, sem0, sem1):
    c = lax.axis_index("c")
    s = lax.axis_index("s")
    wid = c * _NS + s
    pltpu.sync_copy(src_hbm.at[wid, 0], idx_s)
    pltpu.sync_copy(dst_hbm.at[wid, 0], idx_d)
    pltpu.async_copy(h_hbm.at[idx_d.at[0]], rows0, sem0)
    pltpu.sync_copy(zro_hbm, rows1)
    for r in range(_RPW // _CHUNK):
        pltpu.async_copy(rows1, acc.at[pl.ds(s * _RPW + r * _CHUNK, _CHUNK)],
                         sem1)
    for r in range(_RPW // _CHUNK):
        pltpu.make_async_copy(rows1, acc.at[pl.ds(s * _RPW + r * _CHUNK,
                                                  _CHUNK)], sem1).wait()
    plsc.subcore_barrier()

    def gather(j, buf, sem):
        return pltpu.async_copy(h_hbm.at[idx_d.at[j]], buf, sem)

    def gather_wait(j, buf, sem):
        pltpu.make_async_copy(h_hbm.at[idx_d.at[j]], buf, sem).wait()

    def scat(j, buf):
        pltpu.sync_copy(buf, acc.at[idx_s.at[j]], add=True)

    @pl.loop(0, _NSUP)
    def _(p):
        @pl.when(p > 0)
        def _():
            pltpu.sync_copy(src_hbm.at[wid, p], idx_s)
            pltpu.sync_copy(dst_hbm.at[wid, p], idx_d)
            gather(0, rows0, sem0)

        gather(1, rows1, sem1)

        @pl.loop(0, _K - 2, step=2)
        def _(j):
            gather_wait(j, rows0, sem0)
            scat(j, rows0)
            gather(j + 2, rows0, sem0)
            gather_wait(j + 1, rows1, sem1)
            scat(j + 1, rows1)
            gather(j + 3, rows1, sem1)

        gather_wait(_K - 2, rows0, sem0)
        scat(_K - 2, rows0)
        gather_wait(_K - 1, rows1, sem1)
        scat(_K - 1, rows1)

    plsc.subcore_barrier()
    pltpu.sync_copy(acc.at[pl.ds(s * _RPW, _RPW)], out_hbm.at[wid])


_sc_aggr = pl.kernel(
    _sc_aggr_body,
    out_type=jax.ShapeDtypeStruct((_NW, _RPW, _D), jnp.float32),
    mesh=plsc.VectorSubcoreMesh(core_axis_name="c", subcore_axis_name="s"),
    scratch_types=[
        pltpu.VMEM((_K, _CHUNK), jnp.int32),
        pltpu.VMEM((_K, _CHUNK), jnp.int32),
        pltpu.VMEM((_CHUNK, _D), jnp.float32),
        pltpu.VMEM((_CHUNK, _D), jnp.float32),
        pltpu.VMEM_SHARED((_NPAD, _D), jnp.float32),
        pltpu.SemaphoreType.DMA,
        pltpu.SemaphoreType.DMA,
    ],
    name="sc_edge_aggr",
)

_BLK = 2 * _RPW
_GRID = _NPAD // _BLK


def _tc_layer_body(p0, p1, h, wl, w0, w1, b, o):
    aggr = (p0[...] + p1[...]).reshape(_BLK, _D)
    acc = jnp.dot(aggr, wl[...], preferred_element_type=jnp.float32)
    acc += jnp.dot(h[...], w0[...] + w1[...], preferred_element_type=jnp.float32)
    o[...] = jnp.maximum(acc + b[...], 0.0)


def _tc_final_body(p0, p1, h, wl, w0, w1, b, wout, bout, o):
    aggr = (p0[...] + p1[...]).reshape(_BLK, _D)
    acc = jnp.dot(aggr, wl[...], preferred_element_type=jnp.float32)
    acc += jnp.dot(h[...], w0[...] + w1[...], preferred_element_type=jnp.float32)
    hid = jnp.maximum(acc + b[...], 0.0)
    o[...] = jnp.dot(hid, wout[...], preferred_element_type=jnp.float32) + bout[...]


_row_spec = pl.BlockSpec((_BLK, _D), lambda i: (i, 0))
_p0_spec = pl.BlockSpec((2, _RPW, _D), lambda i: (i, 0, 0))
_p1_spec = pl.BlockSpec((2, _RPW, _D), lambda i: (i + _NS // 2, 0, 0))
_w_spec = pl.BlockSpec((_D, _D), lambda i: (0, 0))
_b_spec = pl.BlockSpec((1, _D), lambda i: (0, 0))

_tc_layer = pl.pallas_call(
    _tc_layer_body,
    grid=(_GRID,),
    in_specs=[_p0_spec, _p1_spec, _row_spec, _w_spec, _w_spec, _w_spec, _b_spec],
    out_specs=_row_spec,
    out_shape=jax.ShapeDtypeStruct((_NPAD, _D), jnp.float32),
)

_tc_final = pl.pallas_call(
    _tc_final_body,
    grid=(_GRID,),
    in_specs=[_p0_spec, _p1_spec, _row_spec, _w_spec, _w_spec, _w_spec,
              _b_spec, _w_spec, _b_spec],
    out_specs=_row_spec,
    out_shape=jax.ShapeDtypeStruct((_NPAD, _D), jnp.float32),
)


@jax.jit
def kernel(x, edge_index_0, edge_index_1,
           W0_0, b0_0, Wl_0, bl_0, W1_0, b1_0,
           W0_1, b0_1, Wl_1, bl_1, W1_1, b1_1,
           Wout, bout):
    zeros = jnp.zeros((_CHUNK, _D), jnp.float32)
    xp = jnp.pad(x, ((0, _NPAD - _N), (0, 0)))

    npad = _EPAD - _E
    spad = _N + (jnp.arange(npad, dtype=jnp.int32) % (_NPAD - _N))
    dpad = jnp.arange(npad, dtype=jnp.int32) % _N

    def edge_views(ei):
        src = jnp.concatenate([ei[0], spad]).reshape(_NW, _NSUP, _K, _CHUNK)
        dst = jnp.concatenate([ei[1], dpad]).reshape(_NW, _NSUP, _K, _CHUNK)
        return src, dst

    src1, dst1 = edge_views(edge_index_1)
    src0, dst0 = edge_views(edge_index_0)

    P = _sc_aggr(xp, src1, dst1, zeros)
    h1 = _tc_layer(P, P, xp, Wl_1, W0_1, W1_1,
                   (bl_1 + b0_1 + b1_1).reshape(1, _D))

    Q = _sc_aggr(h1, src0, dst0, zeros)
    out = _tc_final(Q, Q, h1, Wl_0, W0_0, W1_0,
                    (bl_0 + b0_0 + b1_0).reshape(1, _D),
                    Wout, bout.reshape(1, _D))
    return out[:_N]


# speedup vs baseline: 1.3069x; 1.0119x over previous
import functools
import jax
import jax.numpy as jnp
from jax import lax
from jax.experimental import pallas as pl
from jax.experimental.pallas import tpu as pltpu
from jax.experimental.pallas import tpu_sc as plsc

_N = 10000
_E = 320000
_D = 128

_NC = 2
_NS = 16
_NW = _NC * _NS
_CHUNK = 128
_EPW = 10240
_EPAD = _NW * _EPW
_NCHUNK = _EPW // _CHUNK
_K = 40
_NSUP = _NCHUNK // _K
_NPAD = 10240
_RPW = _NPAD // _NS


def _sc_aggr_body(h_hbm, src_hbm, dst_hbm, zro_hbm, out_hbm,
                  idx_s, idx_d, rows0, rows1, acc, sem0, sem1):
    c = lax.axis_index("c")
    s = lax.axis_index("s")
    wid = c * _NS + s
    pltpu.sync_copy(src_hbm.at[wid, 0], idx_s)
    pltpu.sync_copy(dst_hbm.at[wid, 0], idx_d)
    pltpu.async_copy(h_hbm.at[idx_d.at[0]], rows0, sem0)
    pltpu.sync_copy(zro_hbm, rows1)
    for r in range(_RPW // _CHUNK):
        pltpu.async_copy(rows1, acc.at[pl.ds(s * _RPW + r * _CHUNK, _CHUNK)],
                         sem1)
    for r in range(_RPW // _CHUNK):
        pltpu.make_async_copy(rows1, acc.at[pl.ds(s * _RPW + r * _CHUNK,
                                                  _CHUNK)], sem1).wait()
    plsc.subcore_barrier()

    def gather(j, buf, sem):
        return pltpu.async_copy(h_hbm.at[idx_d.at[j]], buf, sem)

    def gather_wait(j, buf, sem):
        pltpu.make_async_copy(h_hbm.at[idx_d.at[j]], buf, sem).wait()

    def scat(j, buf):
        pltpu.sync_copy(buf, acc.at[idx_s.at[j]], add=True)

    @pl.loop(0, _NSUP)
    def _(p):
        @pl.when(p > 0)
        def _():
            pltpu.sync_copy(src_hbm.at[wid, p], idx_s)
            pltpu.sync_copy(dst_hbm.at[wid, p], idx_d)
            gather(0, rows0, sem0)

        gather(1, rows1, sem1)

        @pl.loop(0, _K - 2, step=2)
        def _(j):
            gather_wait(j, rows0, sem0)
            scat(j, rows0)
            gather(j + 2, rows0, sem0)
            gather_wait(j + 1, rows1, sem1)
            scat(j + 1, rows1)
            gather(j + 3, rows1, sem1)

        gather_wait(_K - 2, rows0, sem0)
        scat(_K - 2, rows0)
        gather_wait(_K - 1, rows1, sem1)
        scat(_K - 1, rows1)

    plsc.subcore_barrier()
    pltpu.sync_copy(acc.at[pl.ds(s * _RPW, _RPW)], out_hbm.at[wid])


_sc_aggr = pl.kernel(
    _sc_aggr_body,
    out_type=jax.ShapeDtypeStruct((_NW, _RPW, _D), jnp.float32),
    mesh=plsc.VectorSubcoreMesh(core_axis_name="c", subcore_axis_name="s"),
    scratch_types=[
        pltpu.VMEM((_K, _CHUNK), jnp.int32),
        pltpu.VMEM((_K, _CHUNK), jnp.int32),
        pltpu.VMEM((_CHUNK, _D), jnp.float32),
        pltpu.VMEM((_CHUNK, _D), jnp.float32),
        pltpu.VMEM_SHARED((_NPAD, _D), jnp.float32),
        pltpu.SemaphoreType.DMA,
        pltpu.SemaphoreType.DMA,
    ],
    name="sc_edge_aggr",
)

_BLK = 4 * _RPW
_GRID = _NPAD // _BLK


def _tc_layer_body(p0, p1, h, wl, w0, w1, b, o):
    aggr = (p0[...] + p1[...]).reshape(_BLK, _D)
    acc = jnp.dot(aggr, wl[...], preferred_element_type=jnp.float32)
    acc += jnp.dot(h[...], w0[...] + w1[...], preferred_element_type=jnp.float32)
    o[...] = jnp.maximum(acc + b[...], 0.0)


def _tc_final_body(p0, p1, h, wl, w0, w1, b, wout, bout, o):
    aggr = (p0[...] + p1[...]).reshape(_BLK, _D)
    acc = jnp.dot(aggr, wl[...], preferred_element_type=jnp.float32)
    acc += jnp.dot(h[...], w0[...] + w1[...], preferred_element_type=jnp.float32)
    hid = jnp.maximum(acc + b[...], 0.0)
    o[...] = jnp.dot(hid, wout[...], preferred_element_type=jnp.float32) + bout[...]


_row_spec = pl.BlockSpec((_BLK, _D), lambda i: (i, 0))
_p0_spec = pl.BlockSpec((4, _RPW, _D), lambda i: (i, 0, 0))
_p1_spec = pl.BlockSpec((4, _RPW, _D), lambda i: (i + _NS // 4, 0, 0))
_w_spec = pl.BlockSpec((_D, _D), lambda i: (0, 0))
_b_spec = pl.BlockSpec((1, _D), lambda i: (0, 0))

_tc_layer = pl.pallas_call(
    _tc_layer_body,
    grid=(_GRID,),
    in_specs=[_p0_spec, _p1_spec, _row_spec, _w_spec, _w_spec, _w_spec, _b_spec],
    out_specs=_row_spec,
    out_shape=jax.ShapeDtypeStruct((_NPAD, _D), jnp.float32),
)

_tc_final = pl.pallas_call(
    _tc_final_body,
    grid=(_GRID,),
    in_specs=[_p0_spec, _p1_spec, _row_spec, _w_spec, _w_spec, _w_spec,
              _b_spec, _w_spec, _b_spec],
    out_specs=_row_spec,
    out_shape=jax.ShapeDtypeStruct((_NPAD, _D), jnp.float32),
)


@jax.jit
def kernel(x, edge_index_0, edge_index_1,
           W0_0, b0_0, Wl_0, bl_0, W1_0, b1_0,
           W0_1, b0_1, Wl_1, bl_1, W1_1, b1_1,
           Wout, bout):
    zeros = jnp.zeros((_CHUNK, _D), jnp.float32)
    xp = jnp.pad(x, ((0, _NPAD - _N), (0, 0)))

    npad = _EPAD - _E
    spad = _N + (jnp.arange(npad, dtype=jnp.int32) % (_NPAD - _N))
    dpad = jnp.arange(npad, dtype=jnp.int32) % _N

    def edge_views(ei):
        src = jnp.concatenate([ei[0], spad]).reshape(_NW, _NSUP, _K, _CHUNK)
        dst = jnp.concatenate([ei[1], dpad]).reshape(_NW, _NSUP, _K, _CHUNK)
        return src, dst

    src1, dst1 = edge_views(edge_index_1)
    src0, dst0 = edge_views(edge_index_0)

    P = _sc_aggr(xp, src1, dst1, zeros)
    h1 = _tc_layer(P, P, xp, Wl_1, W0_1, W1_1,
                   (bl_1 + b0_1 + b1_1).reshape(1, _D))

    Q = _sc_aggr(h1, src0, dst0, zeros)
    out = _tc_final(Q, Q, h1, Wl_0, W0_0, W1_0,
                    (bl_0 + b0_0 + b1_0).reshape(1, _D),
                    Wout, bout.reshape(1, _D))
    return out[:_N]


# speedup vs baseline: 1.3166x; 1.0074x over previous
import functools
import jax
import jax.numpy as jnp
from jax import lax
from jax.experimental import pallas as pl
from jax.experimental.pallas import tpu as pltpu
from jax.experimental.pallas import tpu_sc as plsc

_N = 10000
_E = 320000
_D = 128

_NC = 2
_NS = 16
_NW = _NC * _NS
_CHUNK = 128
_EPW = 10240
_EPAD = _NW * _EPW
_NCHUNK = _EPW // _CHUNK
_K = 40
_NSUP = _NCHUNK // _K
_NPAD = 10240
_RPW = _NPAD // _NS


def _sc_aggr_body(h_hbm, src_hbm, dst_hbm, zro_hbm, out_hbm,
                  idx_s, idx_d, rows0, rows1, acc, sem0, sem1):
    c = lax.axis_index("c")
    s = lax.axis_index("s")
    wid = c * _NS + s
    pltpu.sync_copy(src_hbm.at[wid, 0], idx_s)
    pltpu.sync_copy(dst_hbm.at[wid, 0], idx_d)
    pltpu.async_copy(h_hbm.at[idx_d.at[0]], rows0, sem0)
    pltpu.sync_copy(zro_hbm, rows1)
    for r in range(_RPW // _CHUNK):
        pltpu.async_copy(rows1, acc.at[pl.ds(s * _RPW + r * _CHUNK, _CHUNK)],
                         sem1)
    for r in range(_RPW // _CHUNK):
        pltpu.make_async_copy(rows1, acc.at[pl.ds(s * _RPW + r * _CHUNK,
                                                  _CHUNK)], sem1).wait()
    plsc.subcore_barrier()

    def gather(j, buf, sem):
        return pltpu.async_copy(h_hbm.at[idx_d.at[j]], buf, sem)

    def gather_wait(j, buf, sem):
        pltpu.make_async_copy(h_hbm.at[idx_d.at[j]], buf, sem).wait()

    def scat(j, buf):
        pltpu.sync_copy(buf, acc.at[idx_s.at[j]], add=True)

    @pl.loop(0, _NSUP)
    def _(p):
        @pl.when(p > 0)
        def _():
            pltpu.sync_copy(src_hbm.at[wid, p], idx_s)
            pltpu.sync_copy(dst_hbm.at[wid, p], idx_d)
            gather(0, rows0, sem0)

        gather(1, rows1, sem1)

        @pl.loop(0, _K - 2, step=2)
        def _(j):
            gather_wait(j, rows0, sem0)
            scat(j, rows0)
            gather(j + 2, rows0, sem0)
            gather_wait(j + 1, rows1, sem1)
            scat(j + 1, rows1)
            gather(j + 3, rows1, sem1)

        gather_wait(_K - 2, rows0, sem0)
        scat(_K - 2, rows0)
        gather_wait(_K - 1, rows1, sem1)
        scat(_K - 1, rows1)

    plsc.subcore_barrier()
    pltpu.sync_copy(acc.at[pl.ds(s * _RPW, _RPW)], out_hbm.at[wid])


_sc_aggr = pl.kernel(
    _sc_aggr_body,
    out_type=jax.ShapeDtypeStruct((_NW, _RPW, _D), jnp.float32),
    mesh=plsc.VectorSubcoreMesh(core_axis_name="c", subcore_axis_name="s"),
    scratch_types=[
        pltpu.VMEM((_K, _CHUNK), jnp.int32),
        pltpu.VMEM((_K, _CHUNK), jnp.int32),
        pltpu.VMEM((_CHUNK, _D), jnp.float32),
        pltpu.VMEM((_CHUNK, _D), jnp.float32),
        pltpu.VMEM_SHARED((_NPAD, _D), jnp.float32),
        pltpu.SemaphoreType.DMA,
        pltpu.SemaphoreType.DMA,
    ],
    name="sc_edge_aggr",
)

_BLK = 8 * _RPW
_GRID = _NPAD // _BLK


def _tc_layer_body(p0, p1, h, wl, w0, w1, b, o):
    aggr = (p0[...] + p1[...]).reshape(_BLK, _D)
    acc = jnp.dot(aggr, wl[...], preferred_element_type=jnp.float32)
    acc += jnp.dot(h[...], w0[...] + w1[...], preferred_element_type=jnp.float32)
    o[...] = jnp.maximum(acc + b[...], 0.0)


def _tc_final_body(p0, p1, h, wl, w0, w1, b, wout, bout, o):
    aggr = (p0[...] + p1[...]).reshape(_BLK, _D)
    acc = jnp.dot(aggr, wl[...], preferred_element_type=jnp.float32)
    acc += jnp.dot(h[...], w0[...] + w1[...], preferred_element_type=jnp.float32)
    hid = jnp.maximum(acc + b[...], 0.0)
    o[...] = jnp.dot(hid, wout[...], preferred_element_type=jnp.float32) + bout[...]


_row_spec = pl.BlockSpec((_BLK, _D), lambda i: (i, 0))
_p0_spec = pl.BlockSpec((8, _RPW, _D), lambda i: (i, 0, 0))
_p1_spec = pl.BlockSpec((8, _RPW, _D), lambda i: (i + _NS // 8, 0, 0))
_w_spec = pl.BlockSpec((_D, _D), lambda i: (0, 0))
_b_spec = pl.BlockSpec((1, _D), lambda i: (0, 0))

_tc_layer = pl.pallas_call(
    _tc_layer_body,
    grid=(_GRID,),
    in_specs=[_p0_spec, _p1_spec, _row_spec, _w_spec, _w_spec, _w_spec, _b_spec],
    out_specs=_row_spec,
    out_shape=jax.ShapeDtypeStruct((_NPAD, _D), jnp.float32),
)

_tc_final = pl.pallas_call(
    _tc_final_body,
    grid=(_GRID,),
    in_specs=[_p0_spec, _p1_spec, _row_spec, _w_spec, _w_spec, _w_spec,
              _b_spec, _w_spec, _b_spec],
    out_specs=_row_spec,
    out_shape=jax.ShapeDtypeStruct((_NPAD, _D), jnp.float32),
)


@jax.jit
def kernel(x, edge_index_0, edge_index_1,
           W0_0, b0_0, Wl_0, bl_0, W1_0, b1_0,
           W0_1, b0_1, Wl_1, bl_1, W1_1, b1_1,
           Wout, bout):
    zeros = jnp.zeros((_CHUNK, _D), jnp.float32)
    xp = jnp.pad(x, ((0, _NPAD - _N), (0, 0)))

    npad = _EPAD - _E
    spad = _N + (jnp.arange(npad, dtype=jnp.int32) % (_NPAD - _N))
    dpad = jnp.arange(npad, dtype=jnp.int32) % _N

    def edge_views(ei):
        src = jnp.concatenate([ei[0], spad]).reshape(_NW, _NSUP, _K, _CHUNK)
        dst = jnp.concatenate([ei[1], dpad]).reshape(_NW, _NSUP, _K, _CHUNK)
        return src, dst

    src1, dst1 = edge_views(edge_index_1)
    src0, dst0 = edge_views(edge_index_0)

    P = _sc_aggr(xp, src1, dst1, zeros)
    h1 = _tc_layer(P, P, xp, Wl_1, W0_1, W1_1,
                   (bl_1 + b0_1 + b1_1).reshape(1, _D))

    Q = _sc_aggr(h1, src0, dst0, zeros)
    out = _tc_final(Q, Q, h1, Wl_0, W0_0, W1_0,
                    (bl_0 + b0_0 + b1_0).reshape(1, _D),
                    Wout, bout.reshape(1, _D))
    return out[:_N]
